# baseline (speedup 1.0000x reference)
import jax
import jax.numpy as jnp
from jax.experimental import pallas as pl


def _rvq_body(x_ref, w_in_ref, b_in_ref, w_out_ref, b_out_ref, cb_ref,
              cbt_ref, out_ref, q1_ref, q2_ref, com_ref):
    nq = cb_ref.shape[0]
    k = cb_ref.shape[1]
    rows = x_ref.shape[0]

    w_in_b = w_in_ref[...].astype(jnp.bfloat16)
    h = jnp.dot(x_ref[...].astype(jnp.bfloat16), w_in_b,
                preferred_element_type=jnp.float32) + b_in_ref[...]
    r = h
    qsum = jnp.zeros_like(h)
    loss_sum = jnp.float32(0.0)
    iota = jax.lax.broadcasted_iota(jnp.int32, (rows, k), 1)

    for i in range(nq):
        cbtf = cbt_ref[i]
        cb = cb_ref[i]
        cn = jnp.sum(cbtf * cbtf, axis=0, keepdims=True)
        fn = jnp.sum(r * r, axis=1, keepdims=True)
        p = jnp.dot(r.astype(jnp.bfloat16), cbtf.astype(jnp.bfloat16),
                    preferred_element_type=jnp.float32)
        dist = -(fn - 2.0 * p + cn)
        m = jnp.max(dist, axis=1, keepdims=True)
        idx = jnp.min(jnp.where(dist == m, iota, k), axis=1, keepdims=True)
        onehot = (iota == idx).astype(jnp.float32)
        q = jnp.dot(onehot, cb, preferred_element_type=jnp.float32,
                    precision=jax.lax.Precision.HIGHEST)
        d = q - r
        loss_sum = loss_sum + jnp.sum(d * d)
        if i == 0:
            q1_ref[...] = q
        if i == 1:
            q2_ref[...] = q
        qsum = qsum + q
        r = r - q

    w_out_b = w_out_ref[...].astype(jnp.bfloat16)
    out_ref[...] = jnp.dot(qsum.astype(jnp.bfloat16), w_out_b,
                           preferred_element_type=jnp.float32) + b_out_ref[...]

    @pl.when(pl.program_id(0) == 0)
    def _init():
        com_ref[...] = jnp.zeros_like(com_ref)

    com_ref[...] += loss_sum.reshape(1, 1)


def kernel(x, W_in, b_in, W_out, b_out, codebooks):
    b, t, hid = x.shape
    nq, k, cdim = codebooks.shape
    n = b * t
    rows = 512

    xf = x.reshape(n, hid)
    cbt = jnp.transpose(codebooks, (0, 2, 1))

    out, q1, q2, com = pl.pallas_call(
        _rvq_body,
        grid=(n // rows,),
        in_specs=[
            pl.BlockSpec((rows, hid), lambda i: (i, 0)),
            pl.BlockSpec((hid, cdim), lambda i: (0, 0)),
            pl.BlockSpec((1, cdim), lambda i: (0, 0)),
            pl.BlockSpec((cdim, hid), lambda i: (0, 0)),
            pl.BlockSpec((1, hid), lambda i: (0, 0)),
            pl.BlockSpec((nq, k, cdim), lambda i: (0, 0, 0)),
            pl.BlockSpec((nq, cdim, k), lambda i: (0, 0, 0)),
        ],
        out_specs=[
            pl.BlockSpec((rows, hid), lambda i: (i, 0)),
            pl.BlockSpec((rows, cdim), lambda i: (i, 0)),
            pl.BlockSpec((rows, cdim), lambda i: (i, 0)),
            pl.BlockSpec((1, 1), lambda i: (0, 0)),
        ],
        out_shape=[
            jax.ShapeDtypeStruct((n, hid), jnp.float32),
            jax.ShapeDtypeStruct((n, cdim), jnp.float32),
            jax.ShapeDtypeStruct((n, cdim), jnp.float32),
            jax.ShapeDtypeStruct((1, 1), jnp.float32),
        ],
    )(xf, W_in, b_in.reshape(1, cdim), W_out, b_out.reshape(1, hid),
      codebooks, cbt)

    com_scalar = com[0, 0] / jnp.float32(nq * n * cdim)
    return (out.reshape(b, t, hid),
            q1.reshape(b, t, cdim),
            q2.reshape(b, t, cdim),
            com_scalar)


# speedup vs baseline: 2.7902x; 2.7902x over previous
import jax
import jax.numpy as jnp
from jax.experimental import pallas as pl
from jax.experimental.pallas import tpu as pltpu


def _rvq_body(x_ref, w_in_ref, b_in_ref, w_out_ref, b_out_ref,
              cbcat_ref, cbt_ref, out_ref, q1_ref, q2_ref, com_ref, q3_ref):
    nq = cbt_ref.shape[0]
    k = cbt_ref.shape[2]
    cdim = cbt_ref.shape[1]
    rows = x_ref.shape[0]

    w_in_b = w_in_ref[...].astype(jnp.bfloat16)
    h = jnp.dot(x_ref[...].astype(jnp.bfloat16), w_in_b,
                preferred_element_type=jnp.float32) + b_in_ref[...]
    r = h
    qsum = jnp.zeros_like(h)
    sqacc = jnp.zeros_like(h)
    iota = jax.lax.broadcasted_iota(jnp.int32, (rows, k), 1)

    for i in range(nq):
        cbtf = cbt_ref[i]
        cn = jnp.sum(cbtf * cbtf, axis=0, keepdims=True)
        fn = jnp.sum(r * r, axis=1, keepdims=True)
        p = jnp.dot(r.astype(jnp.bfloat16), cbtf.astype(jnp.bfloat16),
                    preferred_element_type=jnp.float32)
        neg = (fn - 2.0 * p) + cn
        mn = jnp.min(neg, axis=1, keepdims=True)
        idx = jnp.min(jnp.where(neg == mn, iota, k), axis=1, keepdims=True)
        oh = (iota == idx).astype(jnp.bfloat16)
        q3_ref[...] = jnp.dot(oh, cbcat_ref[i],
                              preferred_element_type=jnp.float32)
        q = ((q3_ref[:, 0:cdim] + q3_ref[:, cdim:2 * cdim])
             + q3_ref[:, 2 * cdim:3 * cdim])
        d = q - r
        sqacc = sqacc + d * d
        if i == 0:
            q1_ref[...] = q
        if i == 1:
            q2_ref[...] = q
        qsum = qsum + q
        r = r - q

    w_out_b = w_out_ref[...].astype(jnp.bfloat16)
    out_ref[...] = jnp.dot(qsum.astype(jnp.bfloat16), w_out_b,
                           preferred_element_type=jnp.float32) + b_out_ref[...]

    @pl.when(pl.program_id(0) == 0)
    def _init():
        com_ref[...] = jnp.zeros_like(com_ref)

    com_ref[...] += jnp.sum(sqacc).reshape(1, 1)


def kernel(x, W_in, b_in, W_out, b_out, codebooks):
    b, t, hid = x.shape
    nq, k, cdim = codebooks.shape
    n = b * t
    rows = 1024

    xf = x.reshape(n, hid)
    cbt = jnp.transpose(codebooks, (0, 2, 1))
    bits = jax.lax.bitcast_convert_type(codebooks, jnp.uint32)
    hi_f = jax.lax.bitcast_convert_type(bits & jnp.uint32(0xFFFF0000),
                                        jnp.float32)
    cb_hi = jax.lax.bitcast_convert_type((bits >> 16).astype(jnp.uint16),
                                         jnp.bfloat16)
    rem1 = codebooks - hi_f
    b1 = jax.lax.bitcast_convert_type(rem1, jnp.uint32)
    mid_f = jax.lax.bitcast_convert_type(b1 & jnp.uint32(0xFFFF0000),
                                         jnp.float32)
    cb_mid = jax.lax.bitcast_convert_type((b1 >> 16).astype(jnp.uint16),
                                          jnp.bfloat16)
    rem2 = rem1 - mid_f
    b2 = jax.lax.bitcast_convert_type(rem2, jnp.uint32)
    cb_lo = jax.lax.bitcast_convert_type((b2 >> 16).astype(jnp.uint16),
                                         jnp.bfloat16)
    cbcat = jnp.concatenate([cb_hi, cb_mid, cb_lo], axis=2)

    out, q1, q2, com = pl.pallas_call(
        _rvq_body,
        grid=(n // rows,),
        in_specs=[
            pl.BlockSpec((rows, hid), lambda i: (i, 0)),
            pl.BlockSpec((hid, cdim), lambda i: (0, 0)),
            pl.BlockSpec((1, cdim), lambda i: (0, 0)),
            pl.BlockSpec((cdim, hid), lambda i: (0, 0)),
            pl.BlockSpec((1, hid), lambda i: (0, 0)),
            pl.BlockSpec((nq, k, 3 * cdim), lambda i: (0, 0, 0)),
            pl.BlockSpec((nq, cdim, k), lambda i: (0, 0, 0)),
        ],
        out_specs=[
            pl.BlockSpec((rows, hid), lambda i: (i, 0)),
            pl.BlockSpec((rows, cdim), lambda i: (i, 0)),
            pl.BlockSpec((rows, cdim), lambda i: (i, 0)),
            pl.BlockSpec((1, 1), lambda i: (0, 0)),
        ],
        out_shape=[
            jax.ShapeDtypeStruct((n, hid), jnp.float32),
            jax.ShapeDtypeStruct((n, cdim), jnp.float32),
            jax.ShapeDtypeStruct((n, cdim), jnp.float32),
            jax.ShapeDtypeStruct((1, 1), jnp.float32),
        ],
        scratch_shapes=[pltpu.VMEM((rows, 3 * cdim), jnp.float32)],
    )(xf, W_in, b_in.reshape(1, cdim), W_out, b_out.reshape(1, hid),
      cbcat, cbt)

    com_scalar = com[0, 0] / jnp.float32(nq * n * cdim)
    return (out.reshape(b, t, hid),
            q1.reshape(b, t, cdim),
            q2.reshape(b, t, cdim),
            com_scalar)


# speedup vs baseline: 3.0816x; 1.1045x over previous
import jax
import jax.numpy as jnp
from jax.experimental import pallas as pl
from jax.experimental.pallas import tpu as pltpu


def _rvq_body(x_ref, w_in_ref, b_in_ref, w_out_ref, b_out_ref,
              cbcat_ref, cbt_ref, cbt2_ref, out_ref, q1_ref, q2_ref, com_ref,
              q3_ref):
    nq = cbt_ref.shape[0]
    k = cbt_ref.shape[2]
    cdim = cbt_ref.shape[1]
    rows = x_ref.shape[0]

    w_in_b = w_in_ref[...].astype(jnp.bfloat16)
    h = jnp.dot(x_ref[...].astype(jnp.bfloat16), w_in_b,
                preferred_element_type=jnp.float32) + b_in_ref[...]
    r = h
    qsum = jnp.zeros_like(h)
    sqacc = jnp.zeros_like(h)
    iota = jax.lax.broadcasted_iota(jnp.int32, (rows, k), 1).astype(jnp.float32)

    for i in range(nq):
        cbtf = cbt_ref[i]
        cn = jnp.sum(cbtf * cbtf, axis=0, keepdims=True)
        fn = jnp.sum(r * r, axis=1, keepdims=True)
        p2 = jnp.dot(r.astype(jnp.bfloat16), cbt2_ref[i],
                     preferred_element_type=jnp.float32)
        neg = (fn - p2) + cn
        mn = jnp.min(neg, axis=1, keepdims=True)
        idx = jnp.min(jnp.where(neg == mn, iota, jnp.float32(k)),
                      axis=1, keepdims=True)
        oh = (iota == idx).astype(jnp.bfloat16)
        q3_ref[...] = jnp.dot(oh, cbcat_ref[i],
                              preferred_element_type=jnp.float32)
        q = ((q3_ref[:, 0:cdim] + q3_ref[:, cdim:2 * cdim])
             + q3_ref[:, 2 * cdim:3 * cdim])
        d = q - r
        sqacc = sqacc + d * d
        if i == 0:
            q1_ref[...] = q
        if i == 1:
            q2_ref[...] = q
        qsum = qsum + q
        r = r - q

    w_out_b = w_out_ref[...].astype(jnp.bfloat16)
    out_ref[...] = jnp.dot(qsum.astype(jnp.bfloat16), w_out_b,
                           preferred_element_type=jnp.float32) + b_out_ref[...]

    @pl.when(pl.program_id(0) == 0)
    def _init():
        com_ref[...] = jnp.zeros_like(com_ref)

    com_ref[...] += jnp.sum(sqacc).reshape(1, 1)


def kernel(x, W_in, b_in, W_out, b_out, codebooks):
    b, t, hid = x.shape
    nq, k, cdim = codebooks.shape
    n = b * t
    rows = 1024

    xf = x.reshape(n, hid)
    cbt = jnp.transpose(codebooks, (0, 2, 1))
    bits = jax.lax.bitcast_convert_type(codebooks, jnp.uint32)
    hi_f = jax.lax.bitcast_convert_type(bits & jnp.uint32(0xFFFF0000),
                                        jnp.float32)
    cb_hi = jax.lax.bitcast_convert_type((bits >> 16).astype(jnp.uint16),
                                         jnp.bfloat16)
    rem1 = codebooks - hi_f
    b1 = jax.lax.bitcast_convert_type(rem1, jnp.uint32)
    mid_f = jax.lax.bitcast_convert_type(b1 & jnp.uint32(0xFFFF0000),
                                         jnp.float32)
    cb_mid = jax.lax.bitcast_convert_type((b1 >> 16).astype(jnp.uint16),
                                          jnp.bfloat16)
    rem2 = rem1 - mid_f
    b2 = jax.lax.bitcast_convert_type(rem2, jnp.uint32)
    cb_lo = jax.lax.bitcast_convert_type((b2 >> 16).astype(jnp.uint16),
                                         jnp.bfloat16)
    cbcat = jnp.concatenate([cb_hi, cb_mid, cb_lo], axis=2)

    out, q1, q2, com = pl.pallas_call(
        _rvq_body,
        grid=(n // rows,),
        in_specs=[
            pl.BlockSpec((rows, hid), lambda i: (i, 0)),
            pl.BlockSpec((hid, cdim), lambda i: (0, 0)),
            pl.BlockSpec((1, cdim), lambda i: (0, 0)),
            pl.BlockSpec((cdim, hid), lambda i: (0, 0)),
            pl.BlockSpec((1, hid), lambda i: (0, 0)),
            pl.BlockSpec((nq, k, 3 * cdim), lambda i: (0, 0, 0)),
            pl.BlockSpec((nq, cdim, k), lambda i: (0, 0, 0)),
            pl.BlockSpec((nq, cdim, k), lambda i: (0, 0, 0)),
        ],
        out_specs=[
            pl.BlockSpec((rows, hid), lambda i: (i, 0)),
            pl.BlockSpec((rows, cdim), lambda i: (i, 0)),
            pl.BlockSpec((rows, cdim), lambda i: (i, 0)),
            pl.BlockSpec((1, 1), lambda i: (0, 0)),
        ],
        out_shape=[
            jax.ShapeDtypeStruct((n, hid), jnp.float32),
            jax.ShapeDtypeStruct((n, cdim), jnp.float32),
            jax.ShapeDtypeStruct((n, cdim), jnp.float32),
            jax.ShapeDtypeStruct((1, 1), jnp.float32),
        ],
        scratch_shapes=[pltpu.VMEM((rows, 3 * cdim), jnp.float32)],
    )(xf, W_in, b_in.reshape(1, cdim), W_out, b_out.reshape(1, hid),
      cbcat, cbt, (cbt + cbt).astype(jnp.bfloat16))

    com_scalar = com[0, 0] / jnp.float32(nq * n * cdim)
    return (out.reshape(b, t, hid),
            q1.reshape(b, t, cdim),
            q2.reshape(b, t, cdim),
            com_scalar)


# speedup vs baseline: 3.2144x; 1.0431x over previous
import jax
import jax.numpy as jnp
from jax.experimental import pallas as pl
from jax.experimental.pallas import tpu as pltpu


def _rvq_body(x_ref, w_in_ref, b_in_ref, w_out_ref, b_out_ref,
              cbcat_ref, cbt_ref, cbt2_ref, out_ref, q1_ref, q2_ref, com_ref,
              q3_ref):
    nq = cbt_ref.shape[0]
    k = cbt_ref.shape[2]
    cdim = cbt_ref.shape[1]
    rows = x_ref.shape[0]

    w_in_b = w_in_ref[...].astype(jnp.bfloat16)
    h = jnp.dot(x_ref[...].astype(jnp.bfloat16), w_in_b,
                preferred_element_type=jnp.float32) + b_in_ref[...]
    r = h
    qsum = jnp.zeros_like(h)
    sqacc = jnp.zeros_like(h)
    iota = jax.lax.broadcasted_iota(jnp.int32, (rows, k), 1).astype(jnp.float32)

    for i in range(nq):
        cbtf = cbt_ref[i]
        cn = jnp.sum(cbtf * cbtf, axis=0, keepdims=True)
        fn = jnp.sum(r * r, axis=1, keepdims=True)
        p2 = jnp.dot(r.astype(jnp.bfloat16), cbt2_ref[i],
                     preferred_element_type=jnp.float32)
        neg = (fn - p2) + cn
        mn = jnp.min(neg, axis=1, keepdims=True)
        idx = jnp.min(jnp.where(neg == mn, iota, jnp.float32(k)),
                      axis=1, keepdims=True)
        oh = (iota == idx).astype(jnp.bfloat16)
        q3_ref[...] = jnp.dot(oh, cbcat_ref[i],
                              preferred_element_type=jnp.float32)
        q = ((q3_ref[:, 0:cdim] + q3_ref[:, cdim:2 * cdim])
             + q3_ref[:, 2 * cdim:3 * cdim])
        d = q - r
        sqacc = sqacc + d * d
        if i == 0:
            q1_ref[...] = q
        if i == 1:
            q2_ref[...] = q
        qsum = qsum + q
        r = r - q

    w_out_b = w_out_ref[...].astype(jnp.bfloat16)
    out_ref[...] = jnp.dot(qsum.astype(jnp.bfloat16), w_out_b,
                           preferred_element_type=jnp.float32) + b_out_ref[...]

    @pl.when(pl.program_id(0) == 0)
    def _init():
        com_ref[...] = jnp.zeros_like(com_ref)

    com_ref[...] += jnp.sum(sqacc).reshape(1, 1)


def kernel(x, W_in, b_in, W_out, b_out, codebooks):
    b, t, hid = x.shape
    nq, k, cdim = codebooks.shape
    n = b * t
    rows = 2048

    xf = x.reshape(n, hid)
    cbt = jnp.transpose(codebooks, (0, 2, 1))
    bits = jax.lax.bitcast_convert_type(codebooks, jnp.uint32)
    hi_f = jax.lax.bitcast_convert_type(bits & jnp.uint32(0xFFFF0000),
                                        jnp.float32)
    cb_hi = jax.lax.bitcast_convert_type((bits >> 16).astype(jnp.uint16),
                                         jnp.bfloat16)
    rem1 = codebooks - hi_f
    b1 = jax.lax.bitcast_convert_type(rem1, jnp.uint32)
    mid_f = jax.lax.bitcast_convert_type(b1 & jnp.uint32(0xFFFF0000),
                                         jnp.float32)
    cb_mid = jax.lax.bitcast_convert_type((b1 >> 16).astype(jnp.uint16),
                                          jnp.bfloat16)
    rem2 = rem1 - mid_f
    b2 = jax.lax.bitcast_convert_type(rem2, jnp.uint32)
    cb_lo = jax.lax.bitcast_convert_type((b2 >> 16).astype(jnp.uint16),
                                         jnp.bfloat16)
    cbcat = jnp.concatenate([cb_hi, cb_mid, cb_lo], axis=2)

    out, q1, q2, com = pl.pallas_call(
        _rvq_body,
        grid=(n // rows,),
        in_specs=[
            pl.BlockSpec((rows, hid), lambda i: (i, 0)),
            pl.BlockSpec((hid, cdim), lambda i: (0, 0)),
            pl.BlockSpec((1, cdim), lambda i: (0, 0)),
            pl.BlockSpec((cdim, hid), lambda i: (0, 0)),
            pl.BlockSpec((1, hid), lambda i: (0, 0)),
            pl.BlockSpec((nq, k, 3 * cdim), lambda i: (0, 0, 0)),
            pl.BlockSpec((nq, cdim, k), lambda i: (0, 0, 0)),
            pl.BlockSpec((nq, cdim, k), lambda i: (0, 0, 0)),
        ],
        out_specs=[
            pl.BlockSpec((rows, hid), lambda i: (i, 0)),
            pl.BlockSpec((rows, cdim), lambda i: (i, 0)),
            pl.BlockSpec((rows, cdim), lambda i: (i, 0)),
            pl.BlockSpec((1, 1), lambda i: (0, 0)),
        ],
        out_shape=[
            jax.ShapeDtypeStruct((n, hid), jnp.float32),
            jax.ShapeDtypeStruct((n, cdim), jnp.float32),
            jax.ShapeDtypeStruct((n, cdim), jnp.float32),
            jax.ShapeDtypeStruct((1, 1), jnp.float32),
        ],
        scratch_shapes=[pltpu.VMEM((rows, 3 * cdim), jnp.float32)],
    )(xf, W_in, b_in.reshape(1, cdim), W_out, b_out.reshape(1, hid),
      cbcat, cbt, (cbt + cbt).astype(jnp.bfloat16))

    com_scalar = com[0, 0] / jnp.float32(nq * n * cdim)
    return (out.reshape(b, t, hid),
            q1.reshape(b, t, cdim),
            q2.reshape(b, t, cdim),
            com_scalar)
